# Initial kernel scaffold; baseline (speedup 1.0000x reference)
import jax
import jax.numpy as jnp
from jax.experimental import pallas as pl


def kernel(x, edge_index, W1, b1, gamma1, beta1, W2, b2, gamma2, beta2):
    raise NotImplementedError("write your pallas kernel here")


# speedup vs baseline: 13.9355x; 13.9355x over previous
import functools

import jax
import jax.numpy as jnp
from jax import lax
from jax.experimental import pallas as pl
from jax.experimental.pallas import tpu as pltpu
from jax.experimental.pallas import tpu_sc as plsc

N = 10000
E = 320000
D = 128
C = 40
CP = 64
EPS = 1e-5
NC, NS = 2, 16
NW = NC * NS
EW = E // NW
K = 80
NCH = EW // K
NPAD = 10240
RPT = NPAD // NS
DW_DEG = 8

_MESH = plsc.VectorSubcoreMesh(core_axis_name="c", subcore_axis_name="s")
_SC_PARAMS = pltpu.CompilerParams(use_tc_tiling_on_sc=False)


@functools.partial(
    pl.kernel,
    out_type=jax.ShapeDtypeStruct((NC, NPAD, DW_DEG), jnp.float32),
    mesh=_MESH,
    scratch_types=[
        pltpu.VMEM_SHARED((NPAD, DW_DEG), jnp.float32),
        pltpu.VMEM((K,), jnp.int32),
        pltpu.VMEM((K, DW_DEG), jnp.float32),
        pltpu.VMEM((K, DW_DEG), jnp.float32),
    ],
    compiler_params=_SC_PARAMS,
)
def _sc_deg(dst_hbm, ones_hbm, zeros_hbm, out_hbm, acc, didx, ones_v, stage):
    c = lax.axis_index("c")
    s = lax.axis_index("s")
    wid = c * NS + s
    pltpu.sync_copy(zeros_hbm, stage)
    pltpu.sync_copy(ones_hbm, ones_v)

    def zb(j, carry):
        pltpu.sync_copy(stage, acc.at[pl.ds(s * RPT + j * K, K)])
        return carry

    lax.fori_loop(0, RPT // K, zb, 0)
    plsc.subcore_barrier()

    eoff = wid * EW

    def body(i, carry):
        pltpu.sync_copy(dst_hbm.at[pl.ds(eoff + i * K, K)], didx)
        pltpu.sync_copy(ones_v, acc.at[didx], add=True)
        return carry

    lax.fori_loop(0, NCH, body, 0)
    plsc.subcore_barrier()

    def outb(j, carry):
        row0 = s * RPT + j * K
        pltpu.sync_copy(acc.at[pl.ds(row0, K)], stage)
        pltpu.sync_copy(stage, out_hbm.at[c, pl.ds(row0, K)])
        return carry

    lax.fori_loop(0, RPT // K, outb, 0)


def _make_sc_agg(dw):
    @functools.partial(
        pl.kernel,
        out_type=jax.ShapeDtypeStruct((NC, NPAD, dw), jnp.float32),
        mesh=_MESH,
        scratch_types=[
            pltpu.VMEM_SHARED((NPAD, dw), jnp.float32),
            pltpu.VMEM((K,), jnp.int32),
            pltpu.VMEM((K,), jnp.int32),
            pltpu.VMEM((K, dw), jnp.float32),
            pltpu.SemaphoreType.DMA,
        ],
        compiler_params=_SC_PARAMS,
    )
    def agg(xs_hbm, src_hbm, dst_hbm, zeros_hbm, out_hbm,
            acc, sidx, didx, msg, sem):
        c = lax.axis_index("c")
        s = lax.axis_index("s")
        wid = c * NS + s
        pltpu.sync_copy(zeros_hbm, msg)

        def zb(j, carry):
            pltpu.sync_copy(msg, acc.at[pl.ds(s * RPT + j * K, K)])
            return carry

        lax.fori_loop(0, RPT // K, zb, 0)
        plsc.subcore_barrier()

        eoff = wid * EW

        def body(i, carry):
            base = eoff + i * K
            pltpu.sync_copy(src_hbm.at[pl.ds(base, K)], sidx)
            pltpu.sync_copy(dst_hbm.at[pl.ds(base, K)], didx)
            pltpu.async_copy(xs_hbm.at[sidx], msg, sem).wait()
            pltpu.sync_copy(msg, acc.at[didx], add=True)
            return carry

        lax.fori_loop(0, NCH, body, 0)
        plsc.subcore_barrier()

        def outb(j, carry):
            row0 = s * RPT + j * K
            pltpu.sync_copy(acc.at[pl.ds(row0, K)], msg)
            pltpu.sync_copy(msg, out_hbm.at[c, pl.ds(row0, K)])
            return carry

        lax.fori_loop(0, RPT // K, outb, 0)

    return agg


_sc_agg_d = _make_sc_agg(D)
_sc_agg_c = _make_sc_agg(CP)


def _tc_pre_body(x_ref, w1_ref, dp_ref, xs1_ref, dinv_ref):
    deg = dp_ref[0, :N, 0:1] + dp_ref[1, :N, 0:1] + 1.0
    dinv = lax.rsqrt(deg)
    xw = jnp.dot(x_ref[...], w1_ref[...], preferred_element_type=jnp.float32)
    xs1_ref[...] = xw * dinv
    dinv_ref[...] = dinv


def _tc_mid_body(s1_ref, xs1_ref, dinv_ref, b1_ref, g1_ref, be1_ref, w2_ref,
                 xs2_ref):
    dinv = dinv_ref[...]
    t = dinv * (s1_ref[0, :N, :] + s1_ref[1, :N, :] + xs1_ref[...]) + b1_ref[...]
    mean = jnp.mean(t, axis=0, keepdims=True)
    ctr = t - mean
    var = jnp.mean(ctr * ctr, axis=0, keepdims=True)
    h = g1_ref[...] * ctr * lax.rsqrt(var + EPS) + be1_ref[...]
    h = jnp.maximum(h, 0.0)
    xw2 = jnp.dot(h, w2_ref[...], preferred_element_type=jnp.float32)
    xs2_ref[...] = xw2 * dinv


def _tc_final_body(s2_ref, xs2_ref, dinv_ref, b2_ref, g2_ref, be2_ref, o_ref):
    dinv = dinv_ref[...]
    t = dinv * (s2_ref[0, :N, :] + s2_ref[1, :N, :] + xs2_ref[...]) + b2_ref[...]
    mean = jnp.mean(t, axis=0, keepdims=True)
    ctr = t - mean
    var = jnp.mean(ctr * ctr, axis=0, keepdims=True)
    o_ref[...] = g2_ref[...] * ctr * lax.rsqrt(var + EPS) + be2_ref[...]


_tc_pre = pl.pallas_call(
    _tc_pre_body,
    out_shape=[
        jax.ShapeDtypeStruct((N, D), jnp.float32),
        jax.ShapeDtypeStruct((N, 1), jnp.float32),
    ],
)

_tc_mid = pl.pallas_call(
    _tc_mid_body,
    out_shape=jax.ShapeDtypeStruct((N, CP), jnp.float32),
)

_tc_final = pl.pallas_call(
    _tc_final_body,
    out_shape=jax.ShapeDtypeStruct((N, CP), jnp.float32),
)


def kernel(x, edge_index, W1, b1, gamma1, beta1, W2, b2, gamma2, beta2):
    src = edge_index[0].astype(jnp.int32)
    dst = edge_index[1].astype(jnp.int32)

    ones8 = jnp.ones((K, DW_DEG), jnp.float32)
    zeros8 = jnp.zeros((K, DW_DEG), jnp.float32)
    dp = _sc_deg(dst, ones8, zeros8)

    xs1, dinv = _tc_pre(x, W1, dp)

    zeros_d = jnp.zeros((K, D), jnp.float32)
    s1 = _sc_agg_d(xs1, src, dst, zeros_d)

    W2p = jnp.pad(W2, ((0, 0), (0, CP - C)))
    xs2 = _tc_mid(s1, xs1, dinv, b1[None, :], gamma1[None, :],
                  beta1[None, :], W2p)

    zeros_c = jnp.zeros((K, CP), jnp.float32)
    s2 = _sc_agg_c(xs2, src, dst, zeros_c)

    b2p = jnp.pad(b2, (0, CP - C))[None, :]
    g2p = jnp.pad(gamma2, (0, CP - C))[None, :]
    be2p = jnp.pad(beta2, (0, CP - C))[None, :]
    out = _tc_final(s2, xs2, dinv, b2p, g2p, be2p)
    return out[:, :C]
